# baseline (speedup 1.0000x reference)
import functools

import jax
import jax.numpy as jnp
from jax import lax
from jax.experimental import pallas as pl
from jax.experimental.pallas import tpu as pltpu
from jax.experimental.pallas import tpu_sc as plsc

_NC = 2
_NS = 16
_NW = _NC * _NS
_LANES = 16


def _pool_body(x_hbm, tab_hbm, h_hbm, idx_v, buf0, buf1, h_v, sem0, sem1,
               *, rpw2, ctx, d, inv):
    wid = lax.axis_index("s") * _NC + lax.axis_index("c")
    base2 = wid * rpw2
    pltpu.sync_copy(x_hbm.at[pl.ds(base2, rpw2)], idx_v)

    def start(r, buf, sem):
        pltpu.make_async_copy(tab_hbm.at[idx_v.at[r]], buf, sem).start()

    def wait(buf, sem):
        pltpu.make_async_copy(tab_hbm.at[idx_v.at[0]], buf, sem).wait()

    def reduce_pair(buf, r):
        for half in range(2):
            lo = half * ctx
            for v in range(d // _LANES):
                sl = pl.ds(v * _LANES, _LANES)
                acc = buf[lo, sl]
                for j in range(1, ctx):
                    acc = acc + buf[lo + j, sl]
                h_v[2 * r + half, sl] = acc * inv

    start(0, buf0, sem0)
    start(1, buf1, sem1)

    def body(i, carry):
        r = 2 * i
        wait(buf0, sem0)
        reduce_pair(buf0, r)
        start(r + 2, buf0, sem0)
        wait(buf1, sem1)
        reduce_pair(buf1, r + 1)
        start(r + 3, buf1, sem1)
        return carry

    lax.fori_loop(0, rpw2 // 2 - 1, body, 0)
    wait(buf0, sem0)
    reduce_pair(buf0, rpw2 - 2)
    wait(buf1, sem1)
    reduce_pair(buf1, rpw2 - 1)

    pltpu.sync_copy(h_v, h_hbm.at[pl.ds(base2 * 2, rpw2 * 2)])


def _pool(x, emb_table):
    b, ctx = x.shape
    _, d = emb_table.shape
    x2 = x.reshape(b // 2, 2 * ctx)
    rpw2 = (b // 2) // _NW
    mesh = plsc.VectorSubcoreMesh(core_axis_name="c", subcore_axis_name="s")
    body = functools.partial(_pool_body, rpw2=rpw2, ctx=ctx, d=d, inv=1.0 / ctx)
    return pl.kernel(
        body,
        out_type=jax.ShapeDtypeStruct((b, d), jnp.float32),
        mesh=mesh,
        scratch_types=[
            pltpu.VMEM((rpw2, 2 * ctx), jnp.int32),
            pltpu.VMEM((2 * ctx, d), jnp.float32),
            pltpu.VMEM((2 * ctx, d), jnp.float32),
            pltpu.VMEM((rpw2 * 2, d), jnp.float32),
            pltpu.SemaphoreType.DMA,
            pltpu.SemaphoreType.DMA,
        ],
    )(x2, emb_table)


def _mm_body(h_ref, w_ref, o_ref):
    res = lax.dot_general(
        h_ref[...].astype(jnp.bfloat16), w_ref[...].astype(jnp.bfloat16),
        dimension_numbers=(((1,), (1,)), ((), ())),
        preferred_element_type=jnp.float32,
    )
    o_ref[...] = res.astype(jnp.bfloat16)


def _project(h, lin_w, bn=1024):
    b, d = h.shape
    v = lin_w.shape[0]
    grid = (pl.cdiv(v, bn),)
    return pl.pallas_call(
        _mm_body,
        grid=grid,
        in_specs=[
            pl.BlockSpec((b, d), lambda j: (0, 0)),
            pl.BlockSpec((bn, d), lambda j: (j, 0)),
        ],
        out_specs=pl.BlockSpec((b, bn), lambda j: (0, j)),
        out_shape=jax.ShapeDtypeStruct((b, v), jnp.bfloat16),
    )(h, lin_w)


def kernel(x, emb_table, lin_w):
    x = x.astype(jnp.int32)
    h = _pool(x, emb_table)
    return _project(h, lin_w).astype(jnp.float32)
